# Initial kernel scaffold; baseline (speedup 1.0000x reference)
import jax
import jax.numpy as jnp
from jax.experimental import pallas as pl


def kernel(token_ids, weight):
    raise NotImplementedError("write your pallas kernel here")


# speedup vs baseline: 2.9737x; 2.9737x over previous
import functools

import jax
import jax.numpy as jnp
from jax import lax
from jax.experimental import pallas as pl
from jax.experimental.pallas import tpu as pltpu
from jax.experimental.pallas import tpu_sc as plsc

NUM_EMB = 100000
DIM = 128

_info = plsc.get_sparse_core_info()
_NC, _NS = _info.num_cores, _info.num_subcores
_NW = _NC * _NS

_B = 4096 * 50
_BPW = _B // _NW
_CHUNK = 128
_NCHUNK = _BPW // _CHUNK


def _make_kernel():
  mesh = plsc.VectorSubcoreMesh(core_axis_name="c", subcore_axis_name="s")

  @functools.partial(
      pl.kernel,
      mesh=mesh,
      out_type=jax.ShapeDtypeStruct((_B, DIM), jnp.float32),
      scratch_types=[
          pltpu.VMEM((_NCHUNK, _CHUNK), jnp.int32),
          pltpu.VMEM((_CHUNK, DIM), jnp.float32),
          pltpu.SemaphoreType.DMA,
      ],
  )
  def emb_kernel(idx_hbm, table_hbm, out_hbm, idx_v, rows_v, sem):
    wid = lax.axis_index("s") * _NC + lax.axis_index("c")
    pltpu.sync_copy(idx_hbm.at[wid], idx_v)
    base = wid * _BPW

    def body(g, carry):
      pltpu.async_copy(table_hbm.at[idx_v.at[g]], rows_v, sem).wait()
      pltpu.sync_copy(rows_v, out_hbm.at[pl.ds(base + g * _CHUNK, _CHUNK)])
      return carry

    lax.fori_loop(0, _NCHUNK, body, 0)

  return emb_kernel


_emb = _make_kernel()


@jax.jit
def kernel(token_ids, weight):
  idx = token_ids.reshape(_NW, _NCHUNK, _CHUNK).astype(jnp.int32)
  out = _emb(idx, weight)
  return out.reshape(token_ids.shape[0], token_ids.shape[1], DIM)


# speedup vs baseline: 3.3314x; 1.1203x over previous
import functools

import jax
import jax.numpy as jnp
from jax import lax
from jax.experimental import pallas as pl
from jax.experimental.pallas import tpu as pltpu
from jax.experimental.pallas import tpu_sc as plsc

NUM_EMB = 100000
DIM = 128

_info = plsc.get_sparse_core_info()
_NC, _NS = _info.num_cores, _info.num_subcores
_NW = _NC * _NS

_B = 4096 * 50
_BPW = _B // _NW
_CHUNK = 128
_NCHUNK = _BPW // _CHUNK


def _make_kernel():
  mesh = plsc.VectorSubcoreMesh(core_axis_name="c", subcore_axis_name="s")

  @functools.partial(
      pl.kernel,
      mesh=mesh,
      out_type=jax.ShapeDtypeStruct((_B, DIM), jnp.float32),
      scratch_types=[
          pltpu.VMEM((_NCHUNK, _CHUNK), jnp.int32),
          pltpu.VMEM((_CHUNK, DIM), jnp.float32),
          pltpu.VMEM((_CHUNK, DIM), jnp.float32),
          pltpu.SemaphoreType.DMA,
          pltpu.SemaphoreType.DMA,
      ],
  )
  def emb_kernel(idx_hbm, table_hbm, out_hbm, idx_v, rows0, rows1, sem0, sem1):
    wid = lax.axis_index("s") * _NC + lax.axis_index("c")
    pltpu.sync_copy(idx_hbm.at[wid], idx_v)
    base = wid * _BPW

    pltpu.async_copy(table_hbm.at[idx_v.at[0]], rows0, sem0)

    def body(i, carry):
      g0 = 2 * i
      pltpu.async_copy(table_hbm.at[idx_v.at[g0 + 1]], rows1, sem1)
      pltpu.make_async_copy(table_hbm.at[idx_v.at[g0]], rows0, sem0).wait()
      pltpu.sync_copy(rows0, out_hbm.at[pl.ds(base + g0 * _CHUNK, _CHUNK)])

      @pl.when(i < _NCHUNK // 2 - 1)
      def _():
        pltpu.async_copy(table_hbm.at[idx_v.at[g0 + 2]], rows0, sem0)

      pltpu.make_async_copy(table_hbm.at[idx_v.at[g0 + 1]], rows1, sem1).wait()
      pltpu.sync_copy(rows1, out_hbm.at[pl.ds(base + (g0 + 1) * _CHUNK, _CHUNK)])
      return carry

    lax.fori_loop(0, _NCHUNK // 2, body, 0)

  return emb_kernel


_emb = _make_kernel()


@jax.jit
def kernel(token_ids, weight):
  idx = token_ids.reshape(_NW, _NCHUNK, _CHUNK).astype(jnp.int32)
  out = _emb(idx, weight)
  return out.reshape(token_ids.shape[0], token_ids.shape[1], DIM)


# speedup vs baseline: 3.3349x; 1.0010x over previous
import functools

import jax
import jax.numpy as jnp
from jax import lax
from jax.experimental import pallas as pl
from jax.experimental.pallas import tpu as pltpu
from jax.experimental.pallas import tpu_sc as plsc

NUM_EMB = 100000
DIM = 128

_info = plsc.get_sparse_core_info()
_NC, _NS = _info.num_cores, _info.num_subcores
_NW = _NC * _NS

_B = 4096 * 50
_BPW = _B // _NW
_CHUNK = 128
_NCHUNK = _BPW // _CHUNK
_NBUF = 5


def _make_kernel():
  mesh = plsc.VectorSubcoreMesh(core_axis_name="c", subcore_axis_name="s")

  @functools.partial(
      pl.kernel,
      mesh=mesh,
      out_type=jax.ShapeDtypeStruct((_B, DIM), jnp.float32),
      scratch_types=(
          [pltpu.VMEM((_NCHUNK, _CHUNK), jnp.int32)]
          + [pltpu.VMEM((_CHUNK, DIM), jnp.float32) for _ in range(_NBUF)]
          + [pltpu.SemaphoreType.DMA for _ in range(2 * _NBUF)]
      ),
  )
  def emb_kernel(idx_hbm, table_hbm, out_hbm, idx_v, *scratch):
    rows = scratch[:_NBUF]
    gsem = scratch[_NBUF:2 * _NBUF]
    ssem = scratch[2 * _NBUF:]
    wid = lax.axis_index("s") * _NC + lax.axis_index("c")
    pltpu.sync_copy(idx_hbm.at[wid], idx_v)
    base = wid * _BPW

    def fire_gather(g, b):
      pltpu.async_copy(table_hbm.at[idx_v.at[g]], rows[b], gsem[b])

    def out_slice(g):
      return out_hbm.at[pl.ds(base + g * _CHUNK, _CHUNK)]

    for b in range(_NBUF - 1):
      fire_gather(b, b)

    def body(i, carry):
      for b in range(_NBUF):
        g = i * _NBUF + b
        pltpu.make_async_copy(table_hbm.at[idx_v.at[g]], rows[b],
                              gsem[b]).wait()
        pltpu.async_copy(rows[b], out_slice(g), ssem[b])
        nb = (b + _NBUF - 1) % _NBUF

        @pl.when(g + _NBUF - 1 < _NCHUNK)
        def _():
          @pl.when(g >= 1)
          def _():
            pltpu.make_async_copy(rows[nb], out_slice(g - 1),
                                  ssem[nb]).wait()
          fire_gather(g + _NBUF - 1, nb)
      return carry

    lax.fori_loop(0, _NCHUNK // _NBUF, body, 0)

    for b in range(_NBUF):
      g = _NCHUNK - _NBUF + b
      pltpu.make_async_copy(rows[g % _NBUF], out_slice(g),
                            ssem[g % _NBUF]).wait()

  return emb_kernel


_emb = _make_kernel()


@jax.jit
def kernel(token_ids, weight):
  idx = token_ids.reshape(_NW, _NCHUNK, _CHUNK).astype(jnp.int32)
  out = _emb(idx, weight)
  return out.reshape(token_ids.shape[0], token_ids.shape[1], DIM)


# speedup vs baseline: 5.8979x; 1.7686x over previous
import functools

import jax
import jax.numpy as jnp
from jax import lax
from jax.experimental import pallas as pl
from jax.experimental.pallas import tpu as pltpu
from jax.experimental.pallas import tpu_sc as plsc

DIM = 128
NSEQ = 4096
SEQLEN = 50
SEQPAD = 56

_info = plsc.get_sparse_core_info()
_NC, _NS = _info.num_cores, _info.num_subcores
_NW = _NC * _NS
_SPW = NSEQ // _NW
_SCHUNK = 8
_NCHUNK = _SPW // _SCHUNK
_NBUF = 2


def _make_kernel():
  mesh = plsc.VectorSubcoreMesh(core_axis_name="c", subcore_axis_name="s")

  @functools.partial(
      pl.kernel,
      mesh=mesh,
      out_type=jax.ShapeDtypeStruct((NSEQ, SEQLEN, DIM), jnp.float32),
      scratch_types=(
          [pltpu.VMEM((_SPW, SEQPAD), jnp.int32)]
          + [pltpu.VMEM((_SCHUNK, SEQLEN, DIM), jnp.float32)
             for _ in range(_NBUF)]
          + [pltpu.SemaphoreType.DMA for _ in range(2 * _NBUF)]
      ),
  )
  def emb_kernel(idx_hbm, table_hbm, out_hbm, idx_v, *scratch):
    rows = scratch[:_NBUF]
    gsem = scratch[_NBUF:2 * _NBUF]
    ssem = scratch[2 * _NBUF:]
    wid = lax.axis_index("s") * _NC + lax.axis_index("c")
    seq0 = wid * _SPW
    pltpu.sync_copy(idx_hbm.at[pl.ds(seq0, _SPW)], idx_v)

    def fire_gathers(c, b):
      for s in range(_SCHUNK):
        idx_ref = idx_v.at[c * _SCHUNK + s, pl.ds(0, SEQLEN)]
        pltpu.async_copy(table_hbm.at[idx_ref], rows[b].at[s], gsem[b])

    def out_slice(c):
      return out_hbm.at[pl.ds(seq0 + c * _SCHUNK, _SCHUNK)]

    def drain_gathers(b):
      pltpu.make_async_copy(out_hbm.at[pl.ds(0, _SCHUNK)], rows[b],
                            gsem[b]).wait()

    def drain_store(c, b):
      pltpu.make_async_copy(rows[b], out_slice(c), ssem[b]).wait()

    fire_gathers(0, 0)

    def body(i, carry):
      for b in range(_NBUF):
        c = _NBUF * i + b
        drain_gathers(b)
        pltpu.async_copy(rows[b], out_slice(c), ssem[b])
        nb = (b + 1) % _NBUF

        @pl.when(c + 1 < _NCHUNK)
        def _():
          @pl.when(c >= 1)
          def _():
            drain_store(c - 1, nb)
          fire_gathers(c + 1, nb)
      return carry

    lax.fori_loop(0, _NCHUNK // _NBUF, body, 0)

    for b in range(_NBUF):
      drain_store(_NCHUNK - _NBUF + b, b)

  return emb_kernel


_emb = _make_kernel()


@jax.jit
def kernel(token_ids, weight):
  idx = jnp.pad(token_ids.astype(jnp.int32),
                ((0, 0), (0, SEQPAD - SEQLEN)))
  return _emb(idx, weight)
